# Initial kernel scaffold; baseline (speedup 1.0000x reference)
import jax
import jax.numpy as jnp
from jax.experimental import pallas as pl


def kernel(x, cluster_centers):
    raise NotImplementedError("write your pallas kernel here")


# speedup vs baseline: 7.6704x; 7.6704x over previous
import jax
import jax.numpy as jnp
from jax.experimental import pallas as pl


def _codebook_kernel(x_ref, c_ref, out_ref):
    xb = x_ref[...]
    cb = c_ref[...]
    scores = jnp.dot(xb, cb, preferred_element_type=jnp.float32,
                     precision=jax.lax.Precision.HIGHEST)
    half_norm = 0.5 * jnp.sum(cb * cb, axis=0, keepdims=True)
    scores = scores - half_norm
    k = scores.shape[1]
    best = jnp.max(scores, axis=1, keepdims=True)
    iota = jax.lax.broadcasted_iota(jnp.int32, scores.shape, 1)
    idx = jnp.min(jnp.where(scores == best, iota, k), axis=1)
    out_ref[...] = idx.reshape(out_ref.shape)


def kernel(x, cluster_centers):
    b, c, h, w = x.shape
    k = cluster_centers.shape[1]
    p = b * h * w
    xp = jnp.transpose(x, (0, 2, 3, 1)).reshape(p, c)
    cc = cluster_centers.reshape(k, c).T

    rows = p // 128
    idx = pl.pallas_call(
        _codebook_kernel,
        out_shape=jax.ShapeDtypeStruct((rows, 128), jnp.int32),
    )(xp, cc)
    return idx.reshape(b, h, w)[:, None]
